# Initial kernel scaffold; baseline (speedup 1.0000x reference)
import jax
import jax.numpy as jnp
from jax.experimental import pallas as pl


def kernel(depth_map, x_ray):
    raise NotImplementedError("write your pallas kernel here")


# speedup vs baseline: 22.0548x; 22.0548x over previous
import numpy as np
import jax
import jax.numpy as jnp
from jax.experimental import pallas as pl
from jax.experimental.pallas import tpu as pltpu

_B = 8
_IN = 512
_N = 128
_BD = 8
_HI = jax.lax.Precision.HIGHEST

_W1 = float(np.exp(-0.5))
_W2 = float(np.exp(-2.0))


def _resize_weights(out_size: int, in_size: int) -> np.ndarray:
    inv_scale = in_size / out_size
    kernel_scale = max(inv_scale, 1.0)
    sample_f = (np.arange(out_size, dtype=np.float64) + 0.5) * inv_scale - 0.5
    x = np.abs(sample_f[None, :] - np.arange(in_size, dtype=np.float64)[:, None])
    w = np.maximum(0.0, 1.0 - x / kernel_scale)
    total = w.sum(axis=0, keepdims=True)
    w = np.where(np.abs(total) > 1e-8, w / total, 0.0)
    return np.ascontiguousarray(w.T).astype(np.float32)


def _poly_coeffs() -> tuple:
    a0 = 1.0 + 2.0 * _W1
    a1 = 1.0 + _W1 + _W2
    a2 = _W1 + _W2
    a3 = _W2
    u = np.array([0.0, 1.0, 4.0, 9.0])
    v = np.array([a0, a1, a2, a3])
    c = np.linalg.solve(np.vander(u, 4, increasing=True), v)
    return tuple(float(x) for x in c)


_C0, _C1, _C2, _C3 = _poly_coeffs()

_WH = _resize_weights(_N, _IN)
_WWT = np.ascontiguousarray(_WH.T)
_BAND = (np.abs(np.arange(_N)[:, None] - np.arange(_N)[None, :]) <= 1
         ).astype(np.float32)


def _resize_body(dep_ref, xray_ref, wh_ref, wwt_ref, xrr_ref, idxf_ref):
    wh = wh_ref[...]
    wwt = wwt_ref[...]
    dep = dep_ref[0]
    xr = xray_ref[0]
    dep_r = jnp.dot(jnp.dot(wh, dep, precision=_HI,
                            preferred_element_type=jnp.float32),
                    wwt, precision=_HI, preferred_element_type=jnp.float32)
    xr_r = jnp.dot(jnp.dot(wh, xr, precision=_HI,
                           preferred_element_type=jnp.float32),
                   wwt, precision=_HI, preferred_element_type=jnp.float32)
    t = (dep_r - 0.0) / 100.0 * 127.0
    idxf_ref[0] = jnp.clip(jnp.floor(t), 0.0, 127.0)
    xrr_ref[0] = xr_r


def _splat_body(xrr_ref, idxf_ref, band_ref, out_ref):
    idxf = idxf_ref[0]
    xr = xrr_ref[0]
    band = band_ref[...]
    corr_lo = (jnp.where(idxf == 0.0, _W1, 0.0)
               + jnp.where(idxf == 1.0, _W2, 0.0))
    corr_hi = (jnp.where(idxf == 127.0, _W1, 0.0)
               + jnp.where(idxf == 126.0, _W2, 0.0))
    d0 = pl.program_id(1) * _BD
    for j in range(_BD):
        d = d0 + j
        kf = d.astype(jnp.float32) - idxf
        u = kf * kf
        wt = ((_C3 * u + _C2) * u + _C1) * u + _C0
        wt = jnp.where(u <= 9.5, wt, 0.0)
        wt = (wt - jnp.where(d == 0, corr_lo, 0.0)
              - jnp.where(d == _N - 1, corr_hi, 0.0))
        p = xr * wt
        hp = jnp.dot(band, p, preferred_element_type=jnp.float32)
        out_ref[0, j] = jnp.dot(hp, band,
                                preferred_element_type=jnp.float32) * (1.0 / 27.0)


def kernel(depth_map, x_ray):
    dep = depth_map.reshape(_B, _IN, _IN)
    xray = x_ray.reshape(_B, _IN, _IN)
    wh = jnp.asarray(_WH)
    wwt = jnp.asarray(_WWT)
    band = jnp.asarray(_BAND)

    xr_r, idxf = pl.pallas_call(
        _resize_body,
        out_shape=[jax.ShapeDtypeStruct((_B, _N, _N), jnp.float32),
                   jax.ShapeDtypeStruct((_B, _N, _N), jnp.float32)],
        grid=(_B,),
        in_specs=[pl.BlockSpec((1, _IN, _IN), lambda b: (b, 0, 0)),
                  pl.BlockSpec((1, _IN, _IN), lambda b: (b, 0, 0)),
                  pl.BlockSpec((_N, _IN), lambda b: (0, 0)),
                  pl.BlockSpec((_IN, _N), lambda b: (0, 0))],
        out_specs=[pl.BlockSpec((1, _N, _N), lambda b: (b, 0, 0)),
                   pl.BlockSpec((1, _N, _N), lambda b: (b, 0, 0))],
        compiler_params=pltpu.CompilerParams(
            dimension_semantics=("parallel",)),
        name="resize_index",
    )(dep, xray, wh, wwt)

    vol = pl.pallas_call(
        _splat_body,
        out_shape=jax.ShapeDtypeStruct((_B, _N, _N, _N), jnp.float32),
        grid=(_B, _N // _BD),
        in_specs=[pl.BlockSpec((1, _N, _N), lambda b, d: (b, 0, 0)),
                  pl.BlockSpec((1, _N, _N), lambda b, d: (b, 0, 0)),
                  pl.BlockSpec((_N, _N), lambda b, d: (0, 0))],
        out_specs=pl.BlockSpec((1, _BD, _N, _N), lambda b, d: (b, d, 0, 0)),
        compiler_params=pltpu.CompilerParams(
            dimension_semantics=("parallel", "parallel")),
        name="splat_pool",
    )(xr_r, idxf, band)

    return vol.reshape(_B, 1, _N, _N, _N)


# speedup vs baseline: 33.5372x; 1.5206x over previous
import numpy as np
import jax
import jax.numpy as jnp
from jax.experimental import pallas as pl
from jax.experimental.pallas import tpu as pltpu

_B = 8
_IN = 512
_N = 128
_BD = 16
_HI = jax.lax.Precision.HIGHEST

_W1 = float(np.exp(-0.5))
_W2 = float(np.exp(-2.0))


def _resize_weights(out_size: int, in_size: int) -> np.ndarray:
    inv_scale = in_size / out_size
    kernel_scale = max(inv_scale, 1.0)
    sample_f = (np.arange(out_size, dtype=np.float64) + 0.5) * inv_scale - 0.5
    x = np.abs(sample_f[None, :] - np.arange(in_size, dtype=np.float64)[:, None])
    w = np.maximum(0.0, 1.0 - x / kernel_scale)
    total = w.sum(axis=0, keepdims=True)
    w = np.where(np.abs(total) > 1e-8, w / total, 0.0)
    return np.ascontiguousarray(w.T).astype(np.float32)


def _poly_coeffs() -> tuple:
    a0 = 1.0 + 2.0 * _W1
    a1 = 1.0 + _W1 + _W2
    a2 = _W1 + _W2
    a3 = _W2
    u = np.array([0.0, 1.0, 4.0, 9.0])
    v = np.array([a0, a1, a2, a3])
    c = np.linalg.solve(np.vander(u, 4, increasing=True), v)
    return tuple(float(x) for x in c)


_C0, _C1, _C2, _C3 = _poly_coeffs()
_Q0, _Q1, _Q2, _Q3 = (_C0 / 27.0, _C1 / 27.0, _C2 / 27.0, _C3 / 27.0)

_WH = _resize_weights(_N, _IN)
_WWT = np.ascontiguousarray(_WH.T)
_BAND = (np.abs(np.arange(_N)[:, None] - np.arange(_N)[None, :]) <= 1
         ).astype(np.float32)


def _resize_body(dep_ref, xray_ref, wh_ref, wwt_ref, xrr_ref, idxf_ref):
    wh = wh_ref[...]
    wwt = wwt_ref[...]
    dep = dep_ref[0]
    xr = xray_ref[0]
    dep_r = jnp.dot(jnp.dot(wh, dep, precision=_HI,
                            preferred_element_type=jnp.float32),
                    wwt, precision=_HI, preferred_element_type=jnp.float32)
    xr_r = jnp.dot(jnp.dot(wh, xr, preferred_element_type=jnp.float32),
                   wwt, preferred_element_type=jnp.float32)
    t = (dep_r - 0.0) / 100.0 * 127.0
    idxf_ref[0] = jnp.clip(jnp.floor(t), 0.0, 127.0)
    xrr_ref[0] = xr_r


def _splat_body(xrr_ref, idxf_ref, band_ref, out_ref):
    idxf = idxf_ref[0]
    xr = xrr_ref[0]
    band = band_ref[...]
    pid = pl.program_id(1)
    d0 = pid * _BD
    for j in range(_BD):
        d = d0 + j
        kf = d.astype(jnp.float32) - idxf
        u = kf * kf
        wt = ((_Q3 * u + _Q2) * u + _Q1) * u + _Q0
        wt = jnp.where(u <= 9.5, wt, 0.0)
        p = xr * wt
        hp = jnp.dot(band, p, preferred_element_type=jnp.float32)
        out_ref[0, j] = jnp.dot(hp, band, preferred_element_type=jnp.float32)

    @pl.when(pid == 0)
    def _fix_lo():
        corr = (jnp.where(idxf == 0.0, _W1 / 27.0, 0.0)
                + jnp.where(idxf == 1.0, _W2 / 27.0, 0.0))
        hp = jnp.dot(band, xr * corr, preferred_element_type=jnp.float32)
        out_ref[0, 0] = out_ref[0, 0] - jnp.dot(
            hp, band, preferred_element_type=jnp.float32)

    @pl.when(pid == _N // _BD - 1)
    def _fix_hi():
        corr = (jnp.where(idxf == 127.0, _W1 / 27.0, 0.0)
                + jnp.where(idxf == 126.0, _W2 / 27.0, 0.0))
        hp = jnp.dot(band, xr * corr, preferred_element_type=jnp.float32)
        out_ref[0, _BD - 1] = out_ref[0, _BD - 1] - jnp.dot(
            hp, band, preferred_element_type=jnp.float32)


def kernel(depth_map, x_ray):
    dep = depth_map.reshape(_B, _IN, _IN)
    xray = x_ray.reshape(_B, _IN, _IN)
    wh = jnp.asarray(_WH)
    wwt = jnp.asarray(_WWT)
    band = jnp.asarray(_BAND)

    xr_r, idxf = pl.pallas_call(
        _resize_body,
        out_shape=[jax.ShapeDtypeStruct((_B, _N, _N), jnp.float32),
                   jax.ShapeDtypeStruct((_B, _N, _N), jnp.float32)],
        grid=(_B,),
        in_specs=[pl.BlockSpec((1, _IN, _IN), lambda b: (b, 0, 0)),
                  pl.BlockSpec((1, _IN, _IN), lambda b: (b, 0, 0)),
                  pl.BlockSpec((_N, _IN), lambda b: (0, 0)),
                  pl.BlockSpec((_IN, _N), lambda b: (0, 0))],
        out_specs=[pl.BlockSpec((1, _N, _N), lambda b: (b, 0, 0)),
                   pl.BlockSpec((1, _N, _N), lambda b: (b, 0, 0))],
        compiler_params=pltpu.CompilerParams(
            dimension_semantics=("parallel",)),
        name="resize_index",
    )(dep, xray, wh, wwt)

    vol = pl.pallas_call(
        _splat_body,
        out_shape=jax.ShapeDtypeStruct((_B, _N, _N, _N), jnp.float32),
        grid=(_B, _N // _BD),
        in_specs=[pl.BlockSpec((1, _N, _N), lambda b, d: (b, 0, 0)),
                  pl.BlockSpec((1, _N, _N), lambda b, d: (b, 0, 0)),
                  pl.BlockSpec((_N, _N), lambda b, d: (0, 0))],
        out_specs=pl.BlockSpec((1, _BD, _N, _N), lambda b, d: (b, d, 0, 0)),
        compiler_params=pltpu.CompilerParams(
            dimension_semantics=("parallel", "parallel")),
        name="splat_pool",
    )(xr_r, idxf, band)

    return vol.reshape(_B, 1, _N, _N, _N)
